# initial kernel scaffold (unmeasured)
import jax
import jax.numpy as jnp
from jax import lax
from jax.experimental import pallas as pl
from jax.experimental.pallas import tpu as pltpu

N_DEV = 8
N_TOK = 2048
D_IN = 512
D_OUT = 1024
N_EXP_LOCAL = 8
CAP = 25
SLOT = 32
N_SLOTS = N_EXP_LOCAL * SLOT
BLK = 256


def kernel(x, router_W, route_idx, expert_W):
    del router_W
    x_bf = x.astype(jnp.bfloat16)
    w_bf = expert_W.astype(jnp.bfloat16)

    def body(x_ref, idx_ref, w_ref, out_ref, comm_ref, send_sems, recv_sems):
        my = lax.axis_index("i")
        left = lax.rem(my + N_DEV - 1, N_DEV)
        right = lax.rem(my + 1, N_DEV)

        barrier = pltpu.get_barrier_semaphore()
        for nbr in (left, right):
            pl.semaphore_signal(barrier, inc=1, device_id=(nbr,),
                                device_id_type=pl.DeviceIdType.MESH)
        pl.semaphore_wait(barrier, 2)

        e = idx_ref[:, :]

        ltri = (lax.broadcasted_iota(jnp.int32, (BLK, BLK), 1)
                < lax.broadcasted_iota(jnp.int32, (BLK, BLK), 0)
                ).astype(jnp.bfloat16)
        prefix = jnp.zeros((1, 64), jnp.float32)
        ranks = []
        for b in range(N_TOK // BLK):
            eb = e[b * BLK:(b + 1) * BLK, :]
            ob = (lax.broadcasted_iota(jnp.int32, (BLK, 64), 1) == eb
                  ).astype(jnp.bfloat16)
            before = lax.dot(ltri, ob, preferred_element_type=jnp.float32)
            obf = ob.astype(jnp.float32)
            ranks.append(jnp.sum((before + prefix) * obf, axis=1,
                                 keepdims=True))
            prefix = prefix + jnp.sum(obf, axis=0, keepdims=True)
        rank = jnp.concatenate(ranks, axis=0)
        keep = rank < float(CAP)

        def slots_for(dev):
            local = e - dev * N_EXP_LOCAL
            mine = (local >= 0) & (local < N_EXP_LOCAL) & keep
            s = local.astype(jnp.float32) * SLOT + rank
            return jnp.where(mine, s, jnp.float32(-1.0))

        slot_cols = lax.broadcasted_iota(jnp.float32, (N_TOK, N_SLOTS), 1)

        q_me = (slot_cols == slots_for(my)).astype(jnp.bfloat16)
        xd = lax.dot_general(q_me, x_ref[:, :], (((0,), (0,)), ((), ())),
                             preferred_element_type=jnp.float32)
        xd = xd.astype(jnp.bfloat16)
        for el in range(N_EXP_LOCAL):
            yb = lax.dot(xd[el * SLOT:(el + 1) * SLOT, :], w_ref[el],
                         preferred_element_type=jnp.float32)
            comm_ref[0, el * SLOT:(el + 1) * SLOT, :] = yb.astype(jnp.bfloat16)

        out_ref[:, :] = lax.dot(q_me, comm_ref[0],
                                preferred_element_type=jnp.float32)

        for h in range(N_DEV - 1):
            rdma = pltpu.make_async_remote_copy(
                src_ref=comm_ref.at[h],
                dst_ref=comm_ref.at[h + 1],
                send_sem=send_sems.at[h],
                recv_sem=recv_sems.at[h],
                device_id=(right,),
                device_id_type=pl.DeviceIdType.MESH,
            )
            rdma.start()
            rdma.wait()
            origin = lax.rem(my + N_DEV - h - 1, N_DEV)
            q_d = (slot_cols == slots_for(origin)).astype(jnp.bfloat16)
            out_ref[:, :] = out_ref[:, :] + lax.dot(
                q_d, comm_ref[h + 1], preferred_element_type=jnp.float32)

    return pl.pallas_call(
        body,
        out_shape=jax.ShapeDtypeStruct((N_TOK, D_OUT), jnp.float32),
        in_specs=[pl.BlockSpec(memory_space=pltpu.VMEM)] * 3,
        out_specs=pl.BlockSpec(memory_space=pltpu.VMEM),
        scratch_shapes=[
            pltpu.VMEM((N_DEV, N_SLOTS, D_OUT), jnp.bfloat16),
            pltpu.SemaphoreType.DMA((N_DEV - 1,)),
            pltpu.SemaphoreType.DMA((N_DEV - 1,)),
        ],
        compiler_params=pltpu.CompilerParams(collective_id=0),
    )(x_bf, route_idx, w_bf)


# baseline (device time: 83785 ns/iter reference)
import jax
import jax.numpy as jnp
from jax import lax
from jax.experimental import pallas as pl
from jax.experimental.pallas import tpu as pltpu

N_DEV = 8
N_TOK = 2048
D_IN = 512
D_OUT = 1024
N_EXP_LOCAL = 8
CAP = 25
SLOT = 32
N_SLOTS = N_EXP_LOCAL * SLOT
BLK = 256


def kernel(x, router_W, route_idx, expert_W):
    del router_W
    x_bf = x.astype(jnp.bfloat16)
    w_bf = expert_W.astype(jnp.bfloat16)

    def body(x_ref, idx_ref, w_ref, out_ref, comm_ref, send_sems, recv_sems):
        my = lax.axis_index("i")
        left = lax.rem(my + N_DEV - 1, N_DEV)
        right = lax.rem(my + 1, N_DEV)

        barrier = pltpu.get_barrier_semaphore()
        for nbr in (left, right):
            pl.semaphore_signal(barrier, inc=1, device_id=(nbr,),
                                device_id_type=pl.DeviceIdType.MESH)
        pl.semaphore_wait(barrier, 2)

        e = idx_ref[:, :]

        ltri = (lax.broadcasted_iota(jnp.int32, (BLK, BLK), 1)
                < lax.broadcasted_iota(jnp.int32, (BLK, BLK), 0)
                ).astype(jnp.bfloat16)
        prefix = jnp.zeros((1, 64), jnp.float32)
        ranks = []
        for b in range(N_TOK // BLK):
            eb = e[b * BLK:(b + 1) * BLK, :]
            ob = (lax.broadcasted_iota(jnp.int32, (BLK, 64), 1) == eb
                  ).astype(jnp.bfloat16)
            before = lax.dot(ltri, ob, preferred_element_type=jnp.float32)
            obf = ob.astype(jnp.float32)
            ranks.append(jnp.sum((before + prefix) * obf, axis=1,
                                 keepdims=True))
            prefix = prefix + jnp.sum(obf, axis=0, keepdims=True)
        rank = jnp.concatenate(ranks, axis=0).astype(jnp.int32)
        keep = rank < CAP

        def slots_for(dev):
            local = e - dev * N_EXP_LOCAL
            mine = (local >= 0) & (local < N_EXP_LOCAL) & keep
            return jnp.where(mine, local * SLOT + rank, -1)

        slot_cols = lax.broadcasted_iota(jnp.int32, (N_TOK, N_SLOTS), 1)

        q_me = (slot_cols == slots_for(my)).astype(jnp.bfloat16)
        xd = lax.dot_general(q_me, x_ref[:, :], (((0,), (0,)), ((), ())),
                             preferred_element_type=jnp.float32)
        xd = xd.astype(jnp.bfloat16)
        for el in range(N_EXP_LOCAL):
            yb = lax.dot(xd[el * SLOT:(el + 1) * SLOT, :], w_ref[el],
                         preferred_element_type=jnp.float32)
            comm_ref[0, el * SLOT:(el + 1) * SLOT, :] = yb.astype(jnp.bfloat16)

        out_ref[:, :] = lax.dot(q_me, comm_ref[0],
                                preferred_element_type=jnp.float32)

        for h in range(N_DEV - 1):
            rdma = pltpu.make_async_remote_copy(
                src_ref=comm_ref.at[h],
                dst_ref=comm_ref.at[h + 1],
                send_sem=send_sems.at[h],
                recv_sem=recv_sems.at[h],
                device_id=(right,),
                device_id_type=pl.DeviceIdType.MESH,
            )
            rdma.start()
            rdma.wait()
            origin = lax.rem(my + N_DEV - h - 1, N_DEV)
            q_d = (slot_cols == slots_for(origin)).astype(jnp.bfloat16)
            out_ref[:, :] = out_ref[:, :] + lax.dot(
                q_d, comm_ref[h + 1], preferred_element_type=jnp.float32)

    return pl.pallas_call(
        body,
        out_shape=jax.ShapeDtypeStruct((N_TOK, D_OUT), jnp.float32),
        in_specs=[pl.BlockSpec(memory_space=pltpu.VMEM)] * 3,
        out_specs=pl.BlockSpec(memory_space=pltpu.VMEM),
        scratch_shapes=[
            pltpu.VMEM((N_DEV, N_SLOTS, D_OUT), jnp.bfloat16),
            pltpu.SemaphoreType.DMA((N_DEV - 1,)),
            pltpu.SemaphoreType.DMA((N_DEV - 1,)),
        ],
        compiler_params=pltpu.CompilerParams(collective_id=0),
    )(x_bf, route_idx, w_bf)


# device time: 51644 ns/iter; 1.6224x vs baseline; 1.6224x over previous
import jax
import jax.numpy as jnp
from jax import lax
from jax.experimental import pallas as pl
from jax.experimental.pallas import tpu as pltpu

N_DEV = 8
N_TOK = 2048
D_IN = 512
D_OUT = 1024
N_EXP_LOCAL = 8
CAP = 25
SLOT = 32
N_SLOTS = N_EXP_LOCAL * SLOT
BLK = 256


def kernel(x, router_W, route_idx, expert_W):
    del router_W
    x_bf = x.astype(jnp.bfloat16)
    w_bf = expert_W.astype(jnp.bfloat16)

    def body(x_ref, idx_ref, w_ref, out_ref, comm_ref, send_sems, recv_sems):
        my = lax.axis_index("i")
        left = lax.rem(my + N_DEV - 1, N_DEV)
        right = lax.rem(my + 1, N_DEV)

        barrier = pltpu.get_barrier_semaphore()
        for nbr in (left, right):
            pl.semaphore_signal(barrier, inc=1, device_id=(nbr,),
                                device_id_type=pl.DeviceIdType.MESH)
        pl.semaphore_wait(barrier, 2)

        e = idx_ref[:, :]

        ltri = (lax.broadcasted_iota(jnp.int32, (BLK, BLK), 1)
                < lax.broadcasted_iota(jnp.int32, (BLK, BLK), 0)
                ).astype(jnp.bfloat16)
        prefix = jnp.zeros((1, 64), jnp.float32)
        ranks = []
        for b in range(N_TOK // BLK):
            eb = e[b * BLK:(b + 1) * BLK, :]
            ob = (lax.broadcasted_iota(jnp.int32, (BLK, 64), 1) == eb
                  ).astype(jnp.bfloat16)
            before = lax.dot(ltri, ob, preferred_element_type=jnp.float32)
            obf = ob.astype(jnp.float32)
            ranks.append(jnp.sum((before + prefix) * obf, axis=1,
                                 keepdims=True))
            prefix = prefix + jnp.sum(obf, axis=0, keepdims=True)
        rank = jnp.concatenate(ranks, axis=0).astype(jnp.int32)
        keep = rank < CAP

        def slots_for(dev):
            local = e - dev * N_EXP_LOCAL
            mine = (local >= 0) & (local < N_EXP_LOCAL) & keep
            return jnp.where(mine, local * SLOT + rank, -1)

        slot_cols = lax.broadcasted_iota(jnp.int32, (N_TOK, N_SLOTS), 1)

        q_me = (slot_cols == slots_for(my)).astype(jnp.bfloat16)
        xd = lax.dot_general(q_me, x_ref[:, :], (((0,), (0,)), ((), ())),
                             preferred_element_type=jnp.float32)
        xd = xd.astype(jnp.bfloat16)
        for el in range(N_EXP_LOCAL):
            yb = lax.dot(xd[el * SLOT:(el + 1) * SLOT, :], w_ref[el],
                         preferred_element_type=jnp.float32)
            comm_ref[0, el * SLOT:(el + 1) * SLOT, :] = yb.astype(jnp.bfloat16)

        N_FWD = 4
        N_BWD = 3

        def fwd_rdma(h):
            return pltpu.make_async_remote_copy(
                src_ref=comm_ref.at[h],
                dst_ref=comm_ref.at[h + 1],
                send_sem=send_sems.at[h],
                recv_sem=recv_sems.at[h],
                device_id=(right,),
                device_id_type=pl.DeviceIdType.MESH,
            )

        def bwd_rdma(h):
            return pltpu.make_async_remote_copy(
                src_ref=comm_ref.at[0 if h == 0 else 4 + h],
                dst_ref=comm_ref.at[5 + h],
                send_sem=send_sems.at[N_FWD + h],
                recv_sem=recv_sems.at[N_FWD + h],
                device_id=(left,),
                device_id_type=pl.DeviceIdType.MESH,
            )

        def combine(slot, origin, first=False):
            q_d = (slot_cols == slots_for(origin)).astype(jnp.bfloat16)
            y = lax.dot(q_d, comm_ref[slot],
                        preferred_element_type=jnp.float32)
            out_ref[:, :] = y if first else out_ref[:, :] + y

        fwd_rdma(0).start()
        bwd_rdma(0).start()
        combine(0, my, first=True)
        for h in range(N_FWD):
            fwd_rdma(h).wait_recv()
            if h + 1 < N_FWD:
                fwd_rdma(h + 1).start()
            if h < N_BWD:
                bwd_rdma(h).wait_recv()
                if h + 1 < N_BWD:
                    bwd_rdma(h + 1).start()
            combine(h + 1, lax.rem(my + N_DEV - h - 1, N_DEV))
            if h < N_BWD:
                combine(5 + h, lax.rem(my + h + 1, N_DEV))
        for h in range(N_FWD):
            fwd_rdma(h).wait_send()
        for h in range(N_BWD):
            bwd_rdma(h).wait_send()

    return pl.pallas_call(
        body,
        out_shape=jax.ShapeDtypeStruct((N_TOK, D_OUT), jnp.float32),
        in_specs=[pl.BlockSpec(memory_space=pltpu.VMEM)] * 3,
        out_specs=pl.BlockSpec(memory_space=pltpu.VMEM),
        scratch_shapes=[
            pltpu.VMEM((N_DEV, N_SLOTS, D_OUT), jnp.bfloat16),
            pltpu.SemaphoreType.DMA((N_DEV - 1,)),
            pltpu.SemaphoreType.DMA((N_DEV - 1,)),
        ],
        compiler_params=pltpu.CompilerParams(collective_id=0),
    )(x_bf, route_idx, w_bf)
